# baseline (speedup 1.0000x reference)
import functools

import jax
import jax.numpy as jnp
from jax import lax
from jax.experimental import pallas as pl
from jax.experimental.pallas import tpu as pltpu
from jax.experimental.pallas import tpu_sc as plsc

NC, NS = 2, 16
NW = NC * NS
NB = 4
RCH = 80
DP = 128


def _sc_embed(idx, table):
    b, h = idx.shape
    v, d = table.shape
    nch = v // RCH
    spw = b // NW
    nchunk = spw // NB
    mesh = plsc.VectorSubcoreMesh(
        core_axis_name="c", subcore_axis_name="s", num_cores=NC, num_subcores=NS
    )

    @functools.partial(
        pl.kernel,
        out_type=(
            jax.ShapeDtypeStruct((b, h, d), jnp.float32),
            jax.ShapeDtypeStruct((v, DP), jnp.float32),
        ),
        mesh=mesh,
        scratch_types=[
            pltpu.VMEM((2, RCH, d), jnp.float32),
            pltpu.VMEM((RCH, DP), jnp.float32),
            pltpu.VMEM((2, NB, h), jnp.int32),
            pltpu.VMEM((2, NB, h, DP), jnp.float32),
            pltpu.VMEM((NB, h, d), jnp.float32),
            pltpu.SemaphoreType.DMA,
            pltpu.SemaphoreType.DMA,
            pltpu.SemaphoreType.DMA,
            pltpu.SemaphoreType.DMA,
            pltpu.SemaphoreType.REGULAR,
        ],
    )
    def run(
        idx_hbm,
        table_hbm,
        out_hbm,
        padded_hbm,
        tv,
        tv128,
        idxv,
        rows,
        outv,
        semr0,
        semr1,
        semg0,
        semg1,
        bar,
    ):
        cid = lax.axis_index("c")
        sid = lax.axis_index("s")
        wid = sid * NC + cid
        rsems = (semr0, semr1)
        gsems = (semg0, semg1)

        def rfire(i, buf):
            ch = wid + i * NW

            @pl.when(ch < nch)
            def _():
                r0 = pl.multiple_of(ch * RCH, RCH)
                pltpu.async_copy(
                    table_hbm.at[pl.ds(r0, RCH)], tv.at[buf], rsems[buf]
                )

        def rdrain(i, buf):
            ch = wid + i * NW

            @pl.when(ch < nch)
            def _():
                r0 = pl.multiple_of(ch * RCH, RCH)
                pltpu.make_async_copy(
                    table_hbm.at[pl.ds(r0, RCH)], tv.at[buf], rsems[buf]
                ).wait()

                @pl.loop(0, RCH // 8)
                def _(g):
                    base = pl.multiple_of(g * 8, 8)
                    for rr in range(8):
                        sv = tv.at[buf, base + rr]
                        dv = tv128.at[base + rr]
                        for k in range(d // 16):
                            dv[pl.ds(16 * k, 16)] = sv[pl.ds(16 * k, 16)]

                pltpu.sync_copy(tv128, padded_hbm.at[pl.ds(r0, RCH)])

        niter = (nch + NW - 1) // NW
        rfire(0, 0)

        @pl.loop(0, (niter + 1) // 2)
        def _(t):
            a = 2 * t
            rfire(a + 1, 1)
            rdrain(a, 0)
            rfire(a + 2, 0)
            rdrain(a + 1, 1)

        plsc.subcore_barrier()
        pltpu.core_barrier(bar, core_axis_name="c")

        base_s = wid * spw

        def gfire(i, buf):
            @pl.when(i < nchunk)
            def _():
                b0 = base_s + i * NB
                pltpu.sync_copy(idx_hbm.at[pl.ds(b0, NB)], idxv.at[buf])
                for j in range(NB):
                    pltpu.async_copy(
                        padded_hbm.at[idxv.at[buf, j]], rows.at[buf, j], gsems[buf]
                    )

        def gdrain(i, buf):
            @pl.when(i < nchunk)
            def _():
                b0 = base_s + i * NB
                for j in range(NB):
                    pltpu.make_async_copy(
                        padded_hbm.at[idxv.at[buf, j]], rows.at[buf, j], gsems[buf]
                    ).wait()

                @pl.loop(0, NB)
                def _(j):
                    rv = rows.at[buf, j]
                    ov = outv.at[j]
                    for r in range(h):
                        for k in range(d // 16):
                            ov[r, pl.ds(16 * k, 16)] = rv[r, pl.ds(16 * k, 16)]

                pltpu.sync_copy(outv, out_hbm.at[pl.ds(b0, NB)])

        gfire(0, 0)

        @pl.loop(0, nchunk // 2)
        def _(t):
            a = 2 * t
            gfire(a + 1, 1)
            gdrain(a, 0)
            gfire(a + 2, 0)
            gdrain(a + 1, 1)

    out, _ = run(idx, table)
    return out


def kernel(table, input):
    idx = input.astype(jnp.int32)
    return _sc_embed(idx, table)


# speedup vs baseline: 1.0530x; 1.0530x over previous
import functools

import jax
import jax.numpy as jnp
from jax import lax
from jax.experimental import pallas as pl
from jax.experimental.pallas import tpu as pltpu
from jax.experimental.pallas import tpu_sc as plsc

NC = 2
NS = 16
NW = NC * NS

NB = 8


def _sc_gather(idx, table):
    b, h = idx.shape
    d = table.shape[1]
    samples_per_w = b // NW
    chunks = samples_per_w // NB

    mesh = plsc.VectorSubcoreMesh(
        core_axis_name="c", subcore_axis_name="s", num_cores=NC, num_subcores=NS
    )

    @functools.partial(
        pl.kernel,
        out_type=jax.ShapeDtypeStruct((b, h, d), jnp.float32),
        mesh=mesh,
        scratch_types=[
            pltpu.VMEM((NB, h), jnp.int32),
            pltpu.VMEM((NB, h, d), jnp.float32),
            pltpu.SemaphoreType.DMA,
        ],
        compiler_params=pltpu.CompilerParams(use_tc_tiling_on_sc=False),
    )
    def run(idx_hbm, table_hbm, out_hbm, idx_v, rows_v, sem):
        wid = lax.axis_index("s") * NC + lax.axis_index("c")
        base = wid * samples_per_w

        @pl.loop(0, chunks)
        def _(i):
            b0 = base + i * NB
            pltpu.sync_copy(idx_hbm.at[pl.ds(b0, NB)], idx_v)
            copies = []
            for j in range(NB):
                copies.append(
                    pltpu.async_copy(table_hbm.at[idx_v.at[j]], rows_v.at[j], sem)
                )
            for c in copies:
                c.wait()
            pltpu.sync_copy(rows_v, out_hbm.at[pl.ds(b0, NB)])

    return run(idx, table)


def kernel(table, input):
    idx = input.astype(jnp.int32)
    return _sc_gather(idx, table)
